# Initial kernel scaffold; baseline (speedup 1.0000x reference)
import jax
import jax.numpy as jnp
from jax.experimental import pallas as pl


def kernel(x, gate_W, w1, b1, w2, b2):
    raise NotImplementedError("write your pallas kernel here")


# speedup vs baseline: 2.3366x; 2.3366x over previous
import jax
import jax.numpy as jnp
from jax.experimental import pallas as pl
from jax.experimental.pallas import tpu as pltpu

BT = 128
NUM_E = 8
TOPK = 2

_CParams = getattr(pltpu, "CompilerParams", None) or getattr(
    pltpu, "TPUCompilerParams"
)


def _gating_kernel(x_ref, gwp_ref, a1_ref, a2_ref, g1_ref, g2_ref, aux_ref):
    x = x_ref[...]
    gwp = gwp_ref[...]
    logits = jax.lax.dot_general(
        x.astype(jnp.bfloat16),
        gwp.astype(jnp.bfloat16),
        (((1,), (1,)), ((), ())),
        preferred_element_type=jnp.float32,
    )
    t, l = logits.shape
    col = jax.lax.broadcasted_iota(jnp.int32, (t, l), 1)
    lm = jnp.where(col < NUM_E, logits, -jnp.inf)
    mx = jnp.max(lm, axis=-1, keepdims=True)
    ex = jnp.exp(lm - mx)
    s = jnp.sum(ex, axis=-1, keepdims=True)
    probs = ex / s
    m1 = jnp.max(probs, axis=-1, keepdims=True)
    a1 = jnp.min(jnp.where(probs >= m1, col, l), axis=-1, keepdims=True)
    p2 = jnp.where(col == a1, -1.0, probs)
    m2 = jnp.max(p2, axis=-1, keepdims=True)
    a2 = jnp.min(jnp.where(p2 >= m2, col, l), axis=-1, keepdims=True)
    s12 = m1 + m2
    a1_ref[...] = a1
    a2_ref[...] = a2
    g1_ref[...] = m1 / s12
    g2_ref[...] = m2 / s12
    oh1 = (col == a1).astype(jnp.float32)
    frac = jnp.sum(oh1, axis=0, keepdims=True) * (1.0 / t)
    meanp = jnp.sum(probs, axis=0, keepdims=True) * (1.0 / t)
    aux_ref[...] = (NUM_E * jnp.sum(frac * meanp)).reshape(1, 1)


def _moe_ffn_kernel(
    eid_ref, rt_ref, rw_ref, xb_ref, w1_ref, b1_ref, w2_ref, b2_ref, out_ref
):
    del eid_ref
    idx = rt_ref[0]
    t = xb_ref.shape[0]
    lanes = jax.lax.broadcasted_iota(jnp.int32, (BT, t), 1)
    oh = (lanes == idx).astype(jnp.bfloat16)
    xg = jax.lax.dot_general(
        oh, xb_ref[...], (((1,), (0,)), ((), ())),
        preferred_element_type=jnp.float32,
    ).astype(jnp.bfloat16)
    h = (
        jax.lax.dot_general(
            xg, w1_ref[0], (((1,), (0,)), ((), ())),
            preferred_element_type=jnp.float32,
        )
        + b1_ref[0]
    )
    h = (h * 0.5 * (1.0 + jax.lax.erf(h * 0.7071067811865476))).astype(
        jnp.bfloat16
    )
    y = (
        jax.lax.dot_general(
            h, w2_ref[0], (((1,), (0,)), ((), ())),
            preferred_element_type=jnp.float32,
        )
        + b2_ref[0]
    )
    out_ref[...] = (y * rw_ref[0]).astype(jnp.bfloat16)


def _combine_kernel(sa_ref, sb_ref, y_ref, out_ref):
    sa = sa_ref[0]
    sb = sb_ref[0]
    ns = y_ref.shape[0]
    lanes = jax.lax.broadcasted_iota(jnp.int32, (BT, ns), 1)
    oh = (lanes == sa).astype(jnp.bfloat16) + (lanes == sb).astype(jnp.bfloat16)
    out_ref[...] = jax.lax.dot_general(
        oh, y_ref[...], (((1,), (0,)), ((), ())),
        preferred_element_type=jnp.float32,
    )


def _route_metadata(a1, a2, g1, g2, nt, ns):
    t = a1.shape[0]
    p = 2 * t
    e_all = jnp.concatenate([a1, a2]).astype(jnp.int32)
    w_all = jnp.concatenate([g1, g2])
    tok = jnp.tile(jnp.arange(t, dtype=jnp.int32), 2)
    order = jnp.argsort(e_all)
    e_s = e_all[order]
    counts = jnp.sum(
        (e_all[:, None] == jnp.arange(NUM_E, dtype=jnp.int32)[None, :]).astype(
            jnp.int32
        ),
        axis=0,
    )
    csum = jnp.cumsum(counts)
    gstart = jnp.concatenate([jnp.zeros(1, jnp.int32), csum[:-1]])
    pcounts = ((counts + BT - 1) // BT) * BT
    pcsum = jnp.cumsum(pcounts)
    pstart = jnp.concatenate([jnp.zeros(1, jnp.int32), pcsum[:-1]])
    j = jnp.arange(p, dtype=jnp.int32)
    slot = pstart[e_s] + (j - gstart[e_s])
    rows_tok = jnp.zeros(ns, jnp.int32).at[slot].set(tok[order])
    rows_w = jnp.zeros(ns, jnp.float32).at[slot].set(w_all[order])
    tile_eid = (
        jnp.sum(
            (jnp.arange(nt, dtype=jnp.int32)[:, None] * BT >= pstart[None, :]).astype(
                jnp.int32
            ),
            axis=1,
        )
        - 1
    ).astype(jnp.int32)
    slot_of_pair = jnp.zeros(p, jnp.int32).at[order].set(slot)
    return rows_tok, rows_w, tile_eid, slot_of_pair[:t], slot_of_pair[t:]


def kernel(x, gate_W, w1, b1, w2, b2):
    b, t, d = x.shape
    e, _, hdim = w1.shape
    x_flat = x.reshape(t, d)
    p = TOPK * t
    nt = p // BT + NUM_E
    ns = nt * BT

    gwp = jnp.zeros((128, d), jnp.float32).at[:e].set(gate_W)
    a1, a2, g1, g2, aux = pl.pallas_call(
        _gating_kernel,
        out_shape=[
            jax.ShapeDtypeStruct((t, 1), jnp.int32),
            jax.ShapeDtypeStruct((t, 1), jnp.int32),
            jax.ShapeDtypeStruct((t, 1), jnp.float32),
            jax.ShapeDtypeStruct((t, 1), jnp.float32),
            jax.ShapeDtypeStruct((1, 1), jnp.float32),
        ],
    )(x_flat, gwp)

    rows_tok, rows_w, tile_eid, slot_a, slot_b = _route_metadata(
        a1[:, 0], a2[:, 0], g1[:, 0], g2[:, 0], nt, ns
    )

    xb = x_flat.astype(jnp.bfloat16)
    w1b = w1.astype(jnp.bfloat16)
    w2b = w2.astype(jnp.bfloat16)

    grid_spec = pltpu.PrefetchScalarGridSpec(
        num_scalar_prefetch=1,
        grid=(nt,),
        in_specs=[
            pl.BlockSpec((1, BT, 1), lambda i, eid: (i, 0, 0)),
            pl.BlockSpec((1, BT, 1), lambda i, eid: (i, 0, 0)),
            pl.BlockSpec((t, d), lambda i, eid: (0, 0)),
            pl.BlockSpec((1, d, hdim), lambda i, eid: (eid[i], 0, 0)),
            pl.BlockSpec((1, 1, hdim), lambda i, eid: (eid[i], 0, 0)),
            pl.BlockSpec((1, hdim, d), lambda i, eid: (eid[i], 0, 0)),
            pl.BlockSpec((1, 1, d), lambda i, eid: (eid[i], 0, 0)),
        ],
        out_specs=pl.BlockSpec((BT, d), lambda i, eid: (i, 0)),
    )
    y_slots = pl.pallas_call(
        _moe_ffn_kernel,
        grid_spec=grid_spec,
        out_shape=jax.ShapeDtypeStruct((ns, d), jnp.bfloat16),
        compiler_params=_CParams(dimension_semantics=("arbitrary",)),
    )(
        tile_eid,
        rows_tok.reshape(nt, BT, 1),
        rows_w.reshape(nt, BT, 1),
        xb,
        w1b,
        b1.reshape(e, 1, hdim),
        w2b,
        b2.reshape(e, 1, d),
    )

    ntt = t // BT
    out_flat = pl.pallas_call(
        _combine_kernel,
        grid=(ntt,),
        in_specs=[
            pl.BlockSpec((1, BT, 1), lambda i: (i, 0, 0)),
            pl.BlockSpec((1, BT, 1), lambda i: (i, 0, 0)),
            pl.BlockSpec((ns, d), lambda i: (0, 0)),
        ],
        out_specs=pl.BlockSpec((BT, d), lambda i: (i, 0)),
        out_shape=jax.ShapeDtypeStruct((t, d), jnp.float32),
        compiler_params=_CParams(dimension_semantics=("arbitrary",)),
    )(
        slot_a.reshape(ntt, BT, 1),
        slot_b.reshape(ntt, BT, 1),
        y_slots,
    )

    return out_flat.reshape(b, t, d), aux[0, 0]


# speedup vs baseline: 2.7999x; 1.1983x over previous
import jax
import jax.numpy as jnp
from jax.experimental import pallas as pl
from jax.experimental.pallas import tpu as pltpu

BT = 128
NUM_E = 8
TOPK = 2

_CParams = getattr(pltpu, "CompilerParams", None) or getattr(
    pltpu, "TPUCompilerParams"
)


def _gating_kernel(x_ref, gwp_ref, a1_ref, a2_ref, g1_ref, g2_ref, aux_ref):
    x = x_ref[...]
    gwp = gwp_ref[...]
    logits = jax.lax.dot_general(
        x.astype(jnp.bfloat16),
        gwp.astype(jnp.bfloat16),
        (((1,), (1,)), ((), ())),
        preferred_element_type=jnp.float32,
    )
    t, l = logits.shape
    col = jax.lax.broadcasted_iota(jnp.int32, (t, l), 1)
    lm = jnp.where(col < NUM_E, logits, -jnp.inf)
    mx = jnp.max(lm, axis=-1, keepdims=True)
    ex = jnp.exp(lm - mx)
    s = jnp.sum(ex, axis=-1, keepdims=True)
    probs = ex / s
    m1 = jnp.max(probs, axis=-1, keepdims=True)
    a1 = jnp.min(jnp.where(probs >= m1, col, l), axis=-1, keepdims=True)
    p2 = jnp.where(col == a1, -1.0, probs)
    m2 = jnp.max(p2, axis=-1, keepdims=True)
    a2 = jnp.min(jnp.where(p2 >= m2, col, l), axis=-1, keepdims=True)
    s12 = m1 + m2
    a1_ref[...] = a1
    a2_ref[...] = a2
    g1_ref[...] = m1 / s12
    g2_ref[...] = m2 / s12
    oh1 = (col == a1).astype(jnp.float32)
    frac = jnp.sum(oh1, axis=0, keepdims=True) * (1.0 / t)
    meanp = jnp.sum(probs, axis=0, keepdims=True) * (1.0 / t)
    aux_ref[...] = (NUM_E * jnp.sum(frac * meanp)).reshape(1, 1)


def _pass_a_kernel(
    eid_ref, val_ref, rt_ref, xb_ref, w1_ref, b1_ref, h_ref, w1b_ref
):
    i = pl.program_id(0)

    @pl.when(val_ref[i] == 1)
    def _():
        first = i == 0
        changed = jnp.logical_or(
            first, eid_ref[i] != eid_ref[jnp.maximum(i - 1, 0)]
        )

        @pl.when(changed)
        def _():
            w1b_ref[...] = w1_ref[0].astype(jnp.bfloat16)

        idx = rt_ref[0]
        t = xb_ref.shape[0]
        lanes = jax.lax.broadcasted_iota(jnp.int32, (BT, t), 1)
        oh = (lanes == idx).astype(jnp.bfloat16)
        xg = jax.lax.dot_general(
            oh, xb_ref[...], (((1,), (0,)), ((), ())),
            preferred_element_type=jnp.float32,
        ).astype(jnp.bfloat16)
        h = (
            jax.lax.dot_general(
                xg, w1b_ref[...], (((1,), (0,)), ((), ())),
                preferred_element_type=jnp.float32,
            )
            + b1_ref[0]
        )
        h_ref[...] = (
            h * 0.5 * (1.0 + jax.lax.erf(h * 0.7071067811865476))
        ).astype(jnp.bfloat16)


def _pass_b_kernel(
    eid_ref, val_ref, rw_ref, h_ref, w2_ref, b2_ref, y_ref, w2b_ref
):
    i = pl.program_id(0)
    valid = val_ref[i] == 1

    @pl.when(valid)
    def _():
        first = i == 0
        changed = jnp.logical_or(
            first, eid_ref[i] != eid_ref[jnp.maximum(i - 1, 0)]
        )

        @pl.when(changed)
        def _():
            w2b_ref[...] = w2_ref[0].astype(jnp.bfloat16)

        y = (
            jax.lax.dot_general(
                h_ref[...], w2b_ref[...], (((1,), (0,)), ((), ())),
                preferred_element_type=jnp.float32,
            )
            + b2_ref[0]
        )
        y_ref[...] = (y * rw_ref[0]).astype(jnp.bfloat16)

    @pl.when(jnp.logical_not(valid))
    def _():
        y_ref[...] = jnp.zeros_like(y_ref)


def _combine_kernel(sa_ref, sb_ref, y_ref, out_ref):
    sa = sa_ref[0]
    sb = sb_ref[0]
    ns = y_ref.shape[0]
    lanes = jax.lax.broadcasted_iota(jnp.int32, (BT, ns), 1)
    oh = (lanes == sa).astype(jnp.bfloat16) + (lanes == sb).astype(jnp.bfloat16)
    out_ref[...] = jax.lax.dot_general(
        oh, y_ref[...], (((1,), (0,)), ((), ())),
        preferred_element_type=jnp.float32,
    )


def _route_metadata(a1, a2, g1, g2, nt, ns):
    t = a1.shape[0]
    p = 2 * t
    e_all = jnp.concatenate([a1, a2]).astype(jnp.int32)
    w_all = jnp.concatenate([g1, g2])
    tok = jnp.tile(jnp.arange(t, dtype=jnp.int32), 2)
    ohp = (e_all[:, None] == jnp.arange(NUM_E, dtype=jnp.int32)[None, :]).astype(
        jnp.int32
    )
    cums = jnp.cumsum(ohp, axis=0)
    counts = cums[-1]
    rank = jnp.take_along_axis(cums, e_all[:, None], 1)[:, 0] - 1
    pcounts = ((counts + BT - 1) // BT) * BT
    pcsum = jnp.cumsum(pcounts)
    pstart = jnp.concatenate([jnp.zeros(1, jnp.int32), pcsum[:-1]])
    slot = pstart[e_all] + rank
    rows_tok = jnp.zeros(ns, jnp.int32).at[slot].set(tok)
    rows_w = jnp.zeros(ns, jnp.float32).at[slot].set(w_all)
    tile_starts = jnp.arange(nt, dtype=jnp.int32) * BT
    tile_eid = (
        jnp.sum((tile_starts[:, None] >= pstart[None, :]).astype(jnp.int32), axis=1)
        - 1
    ).astype(jnp.int32)
    tile_valid = (tile_starts < pcsum[-1]).astype(jnp.int32)
    return rows_tok, rows_w, tile_eid, tile_valid, slot[:t], slot[t:]


def kernel(x, gate_W, w1, b1, w2, b2):
    b, t, d = x.shape
    e, _, hdim = w1.shape
    x_flat = x.reshape(t, d)
    p = TOPK * t
    nt = p // BT + NUM_E
    ns = nt * BT

    gwp = jnp.zeros((128, d), jnp.float32).at[:e].set(gate_W)
    a1, a2, g1, g2, aux = pl.pallas_call(
        _gating_kernel,
        out_shape=[
            jax.ShapeDtypeStruct((t, 1), jnp.int32),
            jax.ShapeDtypeStruct((t, 1), jnp.int32),
            jax.ShapeDtypeStruct((t, 1), jnp.float32),
            jax.ShapeDtypeStruct((t, 1), jnp.float32),
            jax.ShapeDtypeStruct((1, 1), jnp.float32),
        ],
    )(x_flat, gwp)

    rows_tok, rows_w, tile_eid, tile_valid, slot_a, slot_b = _route_metadata(
        a1[:, 0], a2[:, 0], g1[:, 0], g2[:, 0], nt, ns
    )

    xb = x_flat.astype(jnp.bfloat16)

    grid_a = pltpu.PrefetchScalarGridSpec(
        num_scalar_prefetch=2,
        grid=(nt,),
        in_specs=[
            pl.BlockSpec((1, BT, 1), lambda i, eid, val: (i, 0, 0)),
            pl.BlockSpec((t, d), lambda i, eid, val: (0, 0)),
            pl.BlockSpec((1, d, hdim), lambda i, eid, val: (eid[i], 0, 0)),
            pl.BlockSpec((1, 1, hdim), lambda i, eid, val: (eid[i], 0, 0)),
        ],
        out_specs=pl.BlockSpec((BT, hdim), lambda i, eid, val: (i, 0)),
        scratch_shapes=[pltpu.VMEM((d, hdim), jnp.bfloat16)],
    )
    h_slots = pl.pallas_call(
        _pass_a_kernel,
        grid_spec=grid_a,
        out_shape=jax.ShapeDtypeStruct((ns, hdim), jnp.bfloat16),
        compiler_params=_CParams(dimension_semantics=("arbitrary",)),
    )(
        tile_eid,
        tile_valid,
        rows_tok.reshape(nt, BT, 1),
        xb,
        w1,
        b1.reshape(e, 1, hdim),
    )

    grid_b = pltpu.PrefetchScalarGridSpec(
        num_scalar_prefetch=2,
        grid=(nt,),
        in_specs=[
            pl.BlockSpec((1, BT, 1), lambda i, eid, val: (i, 0, 0)),
            pl.BlockSpec((BT, hdim), lambda i, eid, val: (i, 0)),
            pl.BlockSpec((1, hdim, d), lambda i, eid, val: (eid[i], 0, 0)),
            pl.BlockSpec((1, 1, d), lambda i, eid, val: (eid[i], 0, 0)),
        ],
        out_specs=pl.BlockSpec((BT, d), lambda i, eid, val: (i, 0)),
        scratch_shapes=[pltpu.VMEM((hdim, d), jnp.bfloat16)],
    )
    y_slots = pl.pallas_call(
        _pass_b_kernel,
        grid_spec=grid_b,
        out_shape=jax.ShapeDtypeStruct((ns, d), jnp.bfloat16),
        compiler_params=_CParams(dimension_semantics=("arbitrary",)),
    )(
        tile_eid,
        tile_valid,
        rows_w.reshape(nt, BT, 1),
        h_slots,
        w2,
        b2.reshape(e, 1, d),
    )

    ntt = t // BT
    out_flat = pl.pallas_call(
        _combine_kernel,
        grid=(ntt,),
        in_specs=[
            pl.BlockSpec((1, BT, 1), lambda i: (i, 0, 0)),
            pl.BlockSpec((1, BT, 1), lambda i: (i, 0, 0)),
            pl.BlockSpec((ns, d), lambda i: (0, 0)),
        ],
        out_specs=pl.BlockSpec((BT, d), lambda i: (i, 0)),
        out_shape=jax.ShapeDtypeStruct((t, d), jnp.float32),
        compiler_params=_CParams(dimension_semantics=("arbitrary",)),
    )(
        slot_a.reshape(ntt, BT, 1),
        slot_b.reshape(ntt, BT, 1),
        y_slots,
    )

    return out_flat.reshape(b, t, d), aux[0, 0]
